# baseline (speedup 1.0000x reference)
import functools

import jax
import jax.numpy as jnp
from jax import lax
from jax.experimental import pallas as pl
from jax.experimental.pallas import tpu as pltpu
from jax.experimental.pallas import tpu_sc as plsc

T = 2048
DIM = 1024
DFF = 1408
E = 8
TOPK = 2
NPAIR = T * TOPK
BM = 256
BG = 256
NBG = NPAIR // BG
G = NBG + E - 1
NEG = -3.0e38
NT = (((1,), (1,)), ((), ()))

NW = 32
GCH = 64
GNCH = NPAIR // (NW * GCH)
CCH = 32
CNCH = T // (NW * CCH)


def _routing_body(x_ref, wg_ref, bias_ref, tw_ref, idx_ref):
    x = x_ref[...]
    wg = wg_ref[...]
    logits = jax.lax.dot_general(x, wg, NT, preferred_element_type=jnp.float32)
    gw = jax.nn.sigmoid(logits)
    biased = logits + bias_ref[...]
    e_iota = jax.lax.broadcasted_iota(jnp.int32, (T, E), 1)
    m1 = jnp.max(biased, axis=1, keepdims=True)
    i1 = jnp.min(jnp.where(biased == m1, e_iota, E), axis=1, keepdims=True)
    masked = jnp.where(e_iota == i1, NEG, biased)
    m2 = jnp.max(masked, axis=1, keepdims=True)
    i2 = jnp.min(jnp.where(masked == m2, e_iota, E), axis=1, keepdims=True)
    w1 = jnp.sum(jnp.where(e_iota == i1, gw, 0.0), axis=1, keepdims=True)
    w2 = jnp.sum(jnp.where(e_iota == i2, gw, 0.0), axis=1, keepdims=True)
    s = w1 + w2
    tw_ref[...] = jnp.concatenate([w1 / s, w2 / s], axis=1)
    idx_ref[...] = jnp.concatenate([i1, i2], axis=1)


def _routing(x, Wg, expert_bias):
    return pl.pallas_call(
        _routing_body,
        out_shape=(
            jax.ShapeDtypeStruct((T, TOPK), jnp.float32),
            jax.ShapeDtypeStruct((T, TOPK), jnp.int32),
        ),
    )(x, Wg, expert_bias.reshape(1, E))


def _metadata(idx, tw):
    e_ids = idx.reshape(-1)
    order = jnp.argsort(e_ids, stable=True).astype(jnp.int32)
    sort_tok = (order // TOPK).astype(jnp.int32)
    inv = jnp.zeros((NPAIR,), jnp.int32).at[order].set(
        jnp.arange(NPAIR, dtype=jnp.int32))
    pos = inv.reshape(T, TOPK)
    w_sorted = tw.reshape(-1)[order]
    e_sorted = e_ids[order]
    counts = jnp.zeros((E,), jnp.int32).at[e_ids].add(1)
    ends = jnp.cumsum(counts)
    starts = ends - counts
    b_lo = starts // BG
    b_hi = jnp.maximum(ends - 1, 0) // BG
    nblk = jnp.where(counts > 0, b_hi - b_lo + 1, 0)
    cum = jnp.cumsum(nblk)
    g_actual = cum[-1]
    steps = jnp.arange(G, dtype=jnp.int32)
    e_step = jnp.minimum(
        jnp.searchsorted(cum, steps, side="right").astype(jnp.int32), E - 1)
    prev_cum = jnp.where(e_step > 0, cum[jnp.maximum(e_step - 1, 0)], 0)
    blk_step = b_lo[e_step] + (steps - prev_cum)
    valid = steps < g_actual
    last = jnp.maximum(g_actual - 1, 0)
    e_step = jnp.where(valid, e_step, e_step[last])
    blk_step = jnp.where(valid, blk_step, blk_step[last])
    prev_max = jnp.concatenate(
        [jnp.full((1,), -1, jnp.int32), lax.cummax(blk_step)[:-1]])
    first_visit = blk_step > prev_max
    meta = jnp.stack([e_step, blk_step, first_visit.astype(jnp.int32),
                      valid.astype(jnp.int32)])
    return meta, sort_tok, pos, w_sorted, e_sorted


def _sc_gather(tok, xsrc):
    mesh = plsc.VectorSubcoreMesh(core_axis_name="c", subcore_axis_name="s")
    rows_pw = GCH * GNCH

    @functools.partial(
        pl.kernel, mesh=mesh,
        out_type=jax.ShapeDtypeStruct((NPAIR, DIM), jnp.float32),
        scratch_types=[
            pltpu.VMEM((rows_pw,), jnp.int32),
            pltpu.VMEM((GCH, DIM), jnp.float32),
            pltpu.SemaphoreType.DMA,
        ],
    )
    def k(tok_hbm, x_hbm, out_hbm, idx_v, rows_v, sem):
        wid = lax.axis_index("s") * 2 + lax.axis_index("c")
        base = wid * rows_pw
        pltpu.sync_copy(tok_hbm.at[pl.ds(base, rows_pw)], idx_v)
        for c in range(GNCH):
            pltpu.async_copy(
                x_hbm.at[idx_v.at[pl.ds(c * GCH, GCH)]], rows_v, sem).wait()
            pltpu.sync_copy(rows_v, out_hbm.at[pl.ds(base + c * GCH, GCH)])

    return k(tok, xsrc)


def _ffn_block(x, W1, b1, W2, b2, W3, b3):
    h1 = jax.lax.dot_general(x, W1, NT, preferred_element_type=jnp.float32) + b1
    h3 = jax.lax.dot_general(x, W3, NT, preferred_element_type=jnp.float32) + b3
    p = h1 * h3
    h = p * jax.nn.sigmoid(p)
    return jax.lax.dot_general(h, W2, NT, preferred_element_type=jnp.float32) + b2


def _grouped_body(meta_ref, xs_ref, w1_ref, b1_ref, w2_ref, b2_ref,
                  w3_ref, b3_ref, wso_ref, eso_ref, ys_ref):
    i = pl.program_id(0)
    e = meta_ref[0, i]
    fv = meta_ref[2, i]
    valid = meta_ref[3, i]

    @pl.when(valid == 1)
    def _():
        o = _ffn_block(xs_ref[...], w1_ref[0], b1_ref[0], w2_ref[0],
                       b2_ref[0], w3_ref[0], b3_ref[0])
        w = jnp.where(eso_ref[0, 0, :] == e, wso_ref[0, 0, :], 0.0)
        contrib = o * w[:, None]

        @pl.when(fv == 1)
        def _():
            ys_ref[...] = contrib

        @pl.when(fv == 0)
        def _():
            ys_ref[...] += contrib


def _grouped_moe(meta, xs, W1, b1, W2, b2, W3, b3, w_sorted, e_sorted):
    grid_spec = pltpu.PrefetchScalarGridSpec(
        num_scalar_prefetch=1,
        grid=(G,),
        in_specs=[
            pl.BlockSpec((BG, DIM), lambda i, m: (m[1, i], 0)),
            pl.BlockSpec((1, DFF, DIM), lambda i, m: (m[0, i], 0, 0)),
            pl.BlockSpec((1, 1, DFF), lambda i, m: (m[0, i], 0, 0)),
            pl.BlockSpec((1, DIM, DFF), lambda i, m: (m[0, i], 0, 0)),
            pl.BlockSpec((1, 1, DIM), lambda i, m: (m[0, i], 0, 0)),
            pl.BlockSpec((1, DFF, DIM), lambda i, m: (m[0, i], 0, 0)),
            pl.BlockSpec((1, 1, DFF), lambda i, m: (m[0, i], 0, 0)),
            pl.BlockSpec((1, 1, BG), lambda i, m: (m[1, i], 0, 0)),
            pl.BlockSpec((1, 1, BG), lambda i, m: (m[1, i], 0, 0)),
        ],
        out_specs=pl.BlockSpec((BG, DIM), lambda i, m: (m[1, i], 0)),
    )
    return pl.pallas_call(
        _grouped_body,
        grid_spec=grid_spec,
        out_shape=jax.ShapeDtypeStruct((NPAIR, DIM), jnp.float32),
    )(meta, xs, W1, b1.reshape(E, 1, DFF), W2, b2.reshape(E, 1, DIM),
      W3, b3.reshape(E, 1, DFF),
      w_sorted.reshape(NBG, 1, BG), e_sorted.reshape(NBG, 1, BG))


def _sc_combine(pos0, pos1, ys, z):
    mesh = plsc.VectorSubcoreMesh(core_axis_name="c", subcore_axis_name="s")
    rows_pw = CCH * CNCH

    @functools.partial(
        pl.kernel, mesh=mesh,
        out_type=jax.ShapeDtypeStruct((T, DIM), jnp.float32),
        scratch_types=[
            pltpu.VMEM((rows_pw,), jnp.int32),
            pltpu.VMEM((rows_pw,), jnp.int32),
            pltpu.VMEM((CCH, DIM), jnp.float32),
            pltpu.VMEM((CCH, DIM), jnp.float32),
            pltpu.VMEM((CCH, DIM), jnp.float32),
            pltpu.SemaphoreType.DMA,
            pltpu.SemaphoreType.DMA,
            pltpu.SemaphoreType.DMA,
        ],
    )
    def k(p0_hbm, p1_hbm, ys_hbm, z_hbm, out_hbm, i0_v, i1_v,
          r0_v, r1_v, z_v, s0, s1, sz):
        wid = lax.axis_index("s") * 2 + lax.axis_index("c")
        base = wid * rows_pw
        pltpu.sync_copy(p0_hbm.at[pl.ds(base, rows_pw)], i0_v)
        pltpu.sync_copy(p1_hbm.at[pl.ds(base, rows_pw)], i1_v)
        for c in range(CNCH):
            cp0 = pltpu.async_copy(
                ys_hbm.at[i0_v.at[pl.ds(c * CCH, CCH)]], r0_v, s0)
            cp1 = pltpu.async_copy(
                ys_hbm.at[i1_v.at[pl.ds(c * CCH, CCH)]], r1_v, s1)
            cpz = pltpu.async_copy(
                z_hbm.at[pl.ds(base + c * CCH, CCH)], z_v, sz)
            cp0.wait()
            cp1.wait()
            cpz.wait()

            def row(r, carry):
                for cc in range(DIM // 16):
                    sl = pl.ds(cc * 16, 16)
                    r0_v[r, sl] = r0_v[r, sl] + r1_v[r, sl] + z_v[r, sl]
                return carry

            lax.fori_loop(0, CCH, row, 0)
            pltpu.sync_copy(r0_v, out_hbm.at[pl.ds(base + c * CCH, CCH)])

    return k(pos0, pos1, ys, z)


def _shared_body(x_ref, ws1_ref, bs1_ref, ws2_ref, bs2_ref, ws3_ref, bs3_ref,
                 y_ref):
    y_ref[...] = _ffn_block(x_ref[...], ws1_ref[...], bs1_ref[...],
                            ws2_ref[...], bs2_ref[...], ws3_ref[...],
                            bs3_ref[...])


def _shared_ffn(x, Ws1, bs1, Ws2, bs2, Ws3, bs3):
    zmap = lambda b: (0, 0)
    return pl.pallas_call(
        _shared_body,
        grid=(T // BM,),
        in_specs=[
            pl.BlockSpec((BM, DIM), lambda b: (b, 0)),
            pl.BlockSpec((DFF, DIM), zmap),
            pl.BlockSpec((1, DFF), zmap),
            pl.BlockSpec((DIM, DFF), zmap),
            pl.BlockSpec((1, DIM), zmap),
            pl.BlockSpec((DFF, DIM), zmap),
            pl.BlockSpec((1, DFF), zmap),
        ],
        out_specs=pl.BlockSpec((BM, DIM), lambda b: (b, 0)),
        out_shape=jax.ShapeDtypeStruct((T, DIM), jnp.float32),
    )(x, Ws1, bs1.reshape(1, DFF), Ws2, bs2.reshape(1, DIM),
      Ws3, bs3.reshape(1, DFF))


def kernel(x, Wg, expert_bias, W1, b1, W2, b2, W3, b3, Ws1, bs1, Ws2, bs2, Ws3, bs3):
    tw, idx = _routing(x, Wg, expert_bias)
    z = _shared_ffn(x, Ws1, bs1, Ws2, bs2, Ws3, bs3)
    meta, sort_tok, pos, w_sorted, e_sorted = _metadata(idx, tw)
    xs = _sc_gather(sort_tok, x)
    ys = _grouped_moe(meta, xs, W1, b1, W2, b2, W3, b3, w_sorted, e_sorted)
    y = _sc_combine(pos[:, 0], pos[:, 1], ys, z)
    return (y, tw, idx)


# speedup vs baseline: 1.1642x; 1.1642x over previous
import functools

import jax
import jax.numpy as jnp
from jax import lax
from jax.experimental import pallas as pl
from jax.experimental.pallas import tpu as pltpu
from jax.experimental.pallas import tpu_sc as plsc

T = 2048
DIM = 1024
DFF = 1408
E = 8
TOPK = 2
NPAIR = T * TOPK
BM = 256
BG = 256
NBG = NPAIR // BG
G = NBG + E - 1
NEG = -3.0e38
NT = (((1,), (1,)), ((), ()))

NW = 32
WREP = 128
CCH = 32
CNCH = T // (NW * CCH)


def _routing_body(x_ref, wg_ref, bias_ref, tw_ref, idx_ref):
    x = x_ref[...]
    wg = wg_ref[...]
    logits = jax.lax.dot_general(x, wg, NT, preferred_element_type=jnp.float32)
    gw = jax.nn.sigmoid(logits)
    biased = logits + bias_ref[...]
    e_iota = jax.lax.broadcasted_iota(jnp.int32, (T, E), 1)
    m1 = jnp.max(biased, axis=1, keepdims=True)
    i1 = jnp.min(jnp.where(biased == m1, e_iota, E), axis=1, keepdims=True)
    masked = jnp.where(e_iota == i1, NEG, biased)
    m2 = jnp.max(masked, axis=1, keepdims=True)
    i2 = jnp.min(jnp.where(masked == m2, e_iota, E), axis=1, keepdims=True)
    w1 = jnp.sum(jnp.where(e_iota == i1, gw, 0.0), axis=1, keepdims=True)
    w2 = jnp.sum(jnp.where(e_iota == i2, gw, 0.0), axis=1, keepdims=True)
    s = w1 + w2
    tw_ref[...] = jnp.concatenate([w1 / s, w2 / s], axis=1)
    idx_ref[...] = jnp.concatenate([i1, i2], axis=1)


def _routing(x, Wg, expert_bias):
    return pl.pallas_call(
        _routing_body,
        out_shape=(
            jax.ShapeDtypeStruct((T, TOPK), jnp.float32),
            jax.ShapeDtypeStruct((T, TOPK), jnp.int32),
        ),
    )(x, Wg, expert_bias.reshape(1, E))


def _onehot_pick(sel, table):
    k = table.shape[0]
    oh = sel[:, None] == jnp.arange(k, dtype=jnp.int32)[None, :]
    return jnp.sum(jnp.where(oh, table[None, :], 0), axis=1)


def _metadata(idx):
    e_ids = idx.reshape(-1)
    onehot = (e_ids[:, None] == jnp.arange(E, dtype=jnp.int32)[None, :])
    oh32 = onehot.astype(jnp.int32)
    csum = jnp.cumsum(oh32, axis=0)
    counts = csum[-1]
    ends = jnp.cumsum(counts)
    starts = ends - counts
    rank = jnp.sum(jnp.where(onehot, csum - 1, 0), axis=1)
    slot = _onehot_pick(e_ids, starts) + rank
    b_lo = starts // BG
    b_hi = jnp.maximum(ends - 1, 0) // BG
    nblk = jnp.where(counts > 0, b_hi - b_lo + 1, 0)
    cum = jnp.cumsum(nblk)
    g_actual = cum[-1]
    steps = jnp.arange(G, dtype=jnp.int32)
    e_step = jnp.sum((steps[:, None] >= cum[None, :]).astype(jnp.int32), axis=1)
    e_step = jnp.minimum(e_step, E - 1)
    cum0 = jnp.concatenate([jnp.zeros((1,), jnp.int32), cum])
    prev_cum = _onehot_pick(e_step, cum0)
    blk_step = _onehot_pick(e_step, b_lo) + (steps - prev_cum)
    valid = steps < g_actual
    e_step = jnp.where(valid, e_step,
                       jnp.max(jnp.where(valid, e_step, -1)))
    blk_step = jnp.where(valid, blk_step,
                         jnp.max(jnp.where(valid, blk_step, -1)))
    prev_blk = jnp.concatenate([jnp.full((1,), -1, jnp.int32), blk_step[:-1]])
    first_visit = blk_step != prev_blk
    seg_start = _onehot_pick(e_step, starts)
    seg_end = _onehot_pick(e_step, ends)
    meta = jnp.stack([e_step, blk_step, first_visit.astype(jnp.int32),
                      valid.astype(jnp.int32), seg_start, seg_end])
    slot2 = slot.reshape(T, TOPK)
    return meta, slot2[:, 0], slot2[:, 1]


def _sc_dispatch(slot_e, slot_o, xsrc, tw0r, tw1r):
    mesh = plsc.VectorSubcoreMesh(core_axis_name="c", subcore_axis_name="s")
    rows_pw = T // NW

    @functools.partial(
        pl.kernel, mesh=mesh,
        out_type=(
            jax.ShapeDtypeStruct((NPAIR, DIM), jnp.float32),
            jax.ShapeDtypeStruct((NPAIR, WREP), jnp.float32),
        ),
        scratch_types=[
            pltpu.VMEM((rows_pw,), jnp.int32),
            pltpu.VMEM((rows_pw,), jnp.int32),
            pltpu.VMEM((rows_pw, DIM), jnp.float32),
            pltpu.VMEM((rows_pw, WREP), jnp.float32),
            pltpu.VMEM((rows_pw, WREP), jnp.float32),
            pltpu.SemaphoreType.DMA,
            pltpu.SemaphoreType.DMA,
            pltpu.SemaphoreType.DMA,
            pltpu.SemaphoreType.DMA,
        ],
    )
    def k(se_hbm, so_hbm, x_hbm, w0_hbm, w1_hbm, out_hbm, wso_hbm,
          ie_v, io_v, rows_v, w0_v, w1_v, s0, s1, s2, s3):
        wid = lax.axis_index("s") * 2 + lax.axis_index("c")
        base = wid * rows_pw
        pltpu.sync_copy(se_hbm.at[pl.ds(base, rows_pw)], ie_v)
        pltpu.sync_copy(so_hbm.at[pl.ds(base, rows_pw)], io_v)
        pltpu.sync_copy(x_hbm.at[pl.ds(base, rows_pw)], rows_v)
        pltpu.sync_copy(w0_hbm.at[pl.ds(base, rows_pw)], w0_v)
        pltpu.sync_copy(w1_hbm.at[pl.ds(base, rows_pw)], w1_v)
        cp0 = pltpu.async_copy(rows_v, out_hbm.at[ie_v], s0)
        cp1 = pltpu.async_copy(rows_v, out_hbm.at[io_v], s1)
        cp2 = pltpu.async_copy(w0_v, wso_hbm.at[ie_v], s2)
        cp3 = pltpu.async_copy(w1_v, wso_hbm.at[io_v], s3)
        cp0.wait()
        cp1.wait()
        cp2.wait()
        cp3.wait()

    return k(slot_e, slot_o, xsrc, tw0r, tw1r)


def _ffn_block(x, W1, b1, W2, b2, W3, b3):
    h1 = jax.lax.dot_general(x, W1, NT, preferred_element_type=jnp.float32) + b1
    h3 = jax.lax.dot_general(x, W3, NT, preferred_element_type=jnp.float32) + b3
    p = h1 * h3
    h = p * jax.nn.sigmoid(p)
    return jax.lax.dot_general(h, W2, NT, preferred_element_type=jnp.float32) + b2


def _grouped_body(meta_ref, xs_ref, w1_ref, b1_ref, w2_ref, b2_ref,
                  w3_ref, b3_ref, wso_ref, ys_ref):
    i = pl.program_id(0)
    fv = meta_ref[2, i]
    valid = meta_ref[3, i]

    @pl.when(valid == 1)
    def _():
        o = _ffn_block(xs_ref[...], w1_ref[0], b1_ref[0], w2_ref[0],
                       b2_ref[0], w3_ref[0], b3_ref[0])
        row0 = meta_ref[1, i] * BG
        rows = row0 + jax.lax.broadcasted_iota(jnp.int32, (BG, 1), 0)
        inseg = jnp.logical_and(rows >= meta_ref[4, i], rows < meta_ref[5, i])
        w = wso_ref[:, 0:1]
        contrib = jnp.where(inseg, o * w, 0.0)

        @pl.when(fv == 1)
        def _():
            ys_ref[...] = contrib

        @pl.when(fv == 0)
        def _():
            ys_ref[...] += contrib


def _grouped_moe(meta, xs, W1, b1, W2, b2, W3, b3, wso):
    grid_spec = pltpu.PrefetchScalarGridSpec(
        num_scalar_prefetch=1,
        grid=(G,),
        in_specs=[
            pl.BlockSpec((BG, DIM), lambda i, m: (m[1, i], 0)),
            pl.BlockSpec((1, DFF, DIM), lambda i, m: (m[0, i], 0, 0)),
            pl.BlockSpec((1, 1, DFF), lambda i, m: (m[0, i], 0, 0)),
            pl.BlockSpec((1, DIM, DFF), lambda i, m: (m[0, i], 0, 0)),
            pl.BlockSpec((1, 1, DIM), lambda i, m: (m[0, i], 0, 0)),
            pl.BlockSpec((1, DFF, DIM), lambda i, m: (m[0, i], 0, 0)),
            pl.BlockSpec((1, 1, DFF), lambda i, m: (m[0, i], 0, 0)),
            pl.BlockSpec((BG, WREP), lambda i, m: (m[1, i], 0)),
        ],
        out_specs=pl.BlockSpec((BG, DIM), lambda i, m: (m[1, i], 0)),
    )
    return pl.pallas_call(
        _grouped_body,
        grid_spec=grid_spec,
        out_shape=jax.ShapeDtypeStruct((NPAIR, DIM), jnp.float32),
    )(meta, xs, W1, b1.reshape(E, 1, DFF), W2, b2.reshape(E, 1, DIM),
      W3, b3.reshape(E, 1, DFF), wso)


def _sc_combine(pos0, pos1, ys):
    mesh = plsc.VectorSubcoreMesh(core_axis_name="c", subcore_axis_name="s")
    rows_pw = CCH * CNCH

    @functools.partial(
        pl.kernel, mesh=mesh,
        out_type=jax.ShapeDtypeStruct((T, DIM), jnp.float32),
        scratch_types=[
            pltpu.VMEM((rows_pw,), jnp.int32),
            pltpu.VMEM((rows_pw,), jnp.int32),
            pltpu.VMEM((CCH, DIM), jnp.float32),
            pltpu.VMEM((CCH, DIM), jnp.float32),
            pltpu.SemaphoreType.DMA,
            pltpu.SemaphoreType.DMA,
        ],
    )
    def k(p0_hbm, p1_hbm, ys_hbm, out_hbm, i0_v, i1_v, r0_v, r1_v, s0, s1):
        wid = lax.axis_index("s") * 2 + lax.axis_index("c")
        base = wid * rows_pw
        pltpu.sync_copy(p0_hbm.at[pl.ds(base, rows_pw)], i0_v)
        pltpu.sync_copy(p1_hbm.at[pl.ds(base, rows_pw)], i1_v)
        for c in range(CNCH):
            cp0 = pltpu.async_copy(
                ys_hbm.at[i0_v.at[pl.ds(c * CCH, CCH)]], r0_v, s0)
            cp1 = pltpu.async_copy(
                ys_hbm.at[i1_v.at[pl.ds(c * CCH, CCH)]], r1_v, s1)
            cp0.wait()
            cp1.wait()

            def row(r, carry):
                for cc in range(DIM // 16):
                    sl = pl.ds(cc * 16, 16)
                    r0_v[r, sl] += r1_v[r, sl]
                return carry

            lax.fori_loop(0, CCH, row, 0)
            pltpu.sync_copy(r0_v, out_hbm.at[pl.ds(base + c * CCH, CCH)])

    return k(pos0, pos1, ys)


def _shared_body(x_ref, ws1_ref, bs1_ref, ws2_ref, bs2_ref, ws3_ref, bs3_ref,
                 ymoe_ref, y_ref):
    z = _ffn_block(x_ref[...], ws1_ref[...], bs1_ref[...], ws2_ref[...],
                   bs2_ref[...], ws3_ref[...], bs3_ref[...])
    y_ref[...] = z + ymoe_ref[...]


def _shared_ffn(x, Ws1, bs1, Ws2, bs2, Ws3, bs3, ymoe):
    zmap = lambda b: (0, 0)
    return pl.pallas_call(
        _shared_body,
        grid=(T // BM,),
        in_specs=[
            pl.BlockSpec((BM, DIM), lambda b: (b, 0)),
            pl.BlockSpec((DFF, DIM), zmap),
            pl.BlockSpec((1, DFF), zmap),
            pl.BlockSpec((DIM, DFF), zmap),
            pl.BlockSpec((1, DIM), zmap),
            pl.BlockSpec((DFF, DIM), zmap),
            pl.BlockSpec((1, DFF), zmap),
            pl.BlockSpec((BM, DIM), lambda b: (b, 0)),
        ],
        out_specs=pl.BlockSpec((BM, DIM), lambda b: (b, 0)),
        out_shape=jax.ShapeDtypeStruct((T, DIM), jnp.float32),
    )(x, Ws1, bs1.reshape(1, DFF), Ws2, bs2.reshape(1, DIM),
      Ws3, bs3.reshape(1, DFF), ymoe)


def kernel(x, Wg, expert_bias, W1, b1, W2, b2, W3, b3, Ws1, bs1, Ws2, bs2, Ws3, bs3):
    tw, idx = _routing(x, Wg, expert_bias)
    meta, slot_e, slot_o = _metadata(idx)
    tw0r = jnp.broadcast_to(tw[:, 0:1], (T, WREP))
    tw1r = jnp.broadcast_to(tw[:, 1:2], (T, WREP))
    xs, wso = _sc_dispatch(slot_e, slot_o, x, tw0r, tw1r)
    ys = _grouped_moe(meta, xs, W1, b1, W2, b2, W3, b3, wso)
    ymoe = _sc_combine(slot_e, slot_o, ys)
    y = _shared_ffn(x, Ws1, bs1, Ws2, bs2, Ws3, bs3, ymoe)
    return (y, tw, idx)


# speedup vs baseline: 1.2106x; 1.0398x over previous
import functools

import jax
import jax.numpy as jnp
from jax import lax
from jax.experimental import pallas as pl
from jax.experimental.pallas import tpu as pltpu
from jax.experimental.pallas import tpu_sc as plsc

T = 2048
DIM = 1024
DFF = 1408
E = 8
TOPK = 2
NPAIR = T * TOPK
BM = 256
BG = 256
NBG = NPAIR // BG
G = NBG + E - 1
NEG = -3.0e38
NT = (((1,), (1,)), ((), ()))

NW = 32
WREP = 128
CCH = 32
CNCH = T // (NW * CCH)


def _csum0(a):
    n = a.shape[0]
    d = 1
    while d < n:
        a = a + jnp.concatenate(
            [jnp.zeros((d, a.shape[1]), a.dtype), a[:n - d]], axis=0)
        d *= 2
    return a


def _csum1(a):
    n = a.shape[1]
    d = 1
    while d < n:
        a = a + jnp.concatenate(
            [jnp.zeros((a.shape[0], d), a.dtype), a[:, :n - d]], axis=1)
        d *= 2
    return a


def _pick(sel_col, table_row):
    k = table_row.shape[1]
    oh = sel_col == jax.lax.broadcasted_iota(jnp.int32, (sel_col.shape[0], k), 1)
    return jnp.sum(jnp.where(oh, table_row, 0), axis=1, keepdims=True)


def _routing_body(x_ref, wg_ref, bias_ref, tw_ref, idx_ref, meta_ref,
                  se_ref, so_ref, tw0r_ref, tw1r_ref):
    x = x_ref[...]
    wg = wg_ref[...]
    logits = jax.lax.dot_general(x, wg, NT, preferred_element_type=jnp.float32)
    gw = jax.nn.sigmoid(logits)
    biased = logits + bias_ref[...]
    e_iota = jax.lax.broadcasted_iota(jnp.int32, (T, E), 1)
    m1 = jnp.max(biased, axis=1, keepdims=True)
    i1 = jnp.min(jnp.where(biased == m1, e_iota, E), axis=1, keepdims=True)
    masked = jnp.where(e_iota == i1, NEG, biased)
    m2 = jnp.max(masked, axis=1, keepdims=True)
    i2 = jnp.min(jnp.where(masked == m2, e_iota, E), axis=1, keepdims=True)
    w1 = jnp.sum(jnp.where(e_iota == i1, gw, 0.0), axis=1, keepdims=True)
    w2 = jnp.sum(jnp.where(e_iota == i2, gw, 0.0), axis=1, keepdims=True)
    s = w1 + w2
    w1n = w1 / s
    w2n = w2 / s
    tw_ref[...] = jnp.concatenate([w1n, w2n], axis=1)
    idx_ref[...] = jnp.concatenate([i1, i2], axis=1)
    tw0r_ref[...] = jnp.broadcast_to(w1n, (T, WREP))
    tw1r_ref[...] = jnp.broadcast_to(w2n, (T, WREP))

    oh0 = (i1 == e_iota).astype(jnp.int32)
    oh1 = (i2 == e_iota).astype(jnp.int32)
    cum0 = _csum0(oh0)
    cum1 = _csum0(oh1)
    rank0 = cum0 + cum1 - oh1 - 1
    rank1 = cum0 + cum1 - 1
    counts = cum0[-1:] + cum1[-1:]
    ends = _csum1(counts)
    starts = ends - counts
    se_ref[...] = jnp.sum(jnp.where(oh0 == 1, starts + rank0, 0),
                          axis=1, keepdims=True)
    so_ref[...] = jnp.sum(jnp.where(oh1 == 1, starts + rank1, 0),
                          axis=1, keepdims=True)

    b_lo = starts // BG
    b_hi = jnp.maximum(ends - 1, 0) // BG
    nblk = jnp.where(counts > 0, b_hi - b_lo + 1, 0)
    cum = _csum1(nblk)
    g_actual = cum[:, -1:]
    steps = jax.lax.broadcasted_iota(jnp.int32, (G, 1), 0)
    e_step = jnp.sum((steps >= cum).astype(jnp.int32), axis=1, keepdims=True)
    e_step = jnp.minimum(e_step, E - 1)
    cumz = jnp.concatenate([jnp.zeros((1, 1), jnp.int32), cum], axis=1)
    prev_cum = _pick(e_step, cumz)
    blk_step = _pick(e_step, b_lo) + (steps - prev_cum)
    valid = steps < g_actual
    e_step = jnp.where(valid, e_step,
                       jnp.max(jnp.where(valid, e_step, -1),
                               axis=0, keepdims=True))
    blk_step = jnp.where(valid, blk_step,
                         jnp.max(jnp.where(valid, blk_step, -1),
                                 axis=0, keepdims=True))
    prev_blk = jnp.concatenate(
        [jnp.full((1, 1), -1, jnp.int32), blk_step[:-1]], axis=0)
    first_visit = (blk_step != prev_blk).astype(jnp.int32)
    meta_ref[...] = jnp.concatenate(
        [e_step, blk_step, first_visit, valid.astype(jnp.int32),
         _pick(e_step, starts), _pick(e_step, ends)], axis=1)


def _routing(x, Wg, expert_bias):
    return pl.pallas_call(
        _routing_body,
        out_shape=(
            jax.ShapeDtypeStruct((T, TOPK), jnp.float32),
            jax.ShapeDtypeStruct((T, TOPK), jnp.int32),
            jax.ShapeDtypeStruct((G, 6), jnp.int32),
            jax.ShapeDtypeStruct((T, 1), jnp.int32),
            jax.ShapeDtypeStruct((T, 1), jnp.int32),
            jax.ShapeDtypeStruct((T, WREP), jnp.float32),
            jax.ShapeDtypeStruct((T, WREP), jnp.float32),
        ),
    )(x, Wg, expert_bias.reshape(1, E))


def _sc_dispatch(slot_e, slot_o, xsrc, tw0r, tw1r):
    mesh = plsc.VectorSubcoreMesh(core_axis_name="c", subcore_axis_name="s")
    rows_pw = T // NW

    @functools.partial(
        pl.kernel, mesh=mesh,
        out_type=(
            jax.ShapeDtypeStruct((NPAIR, DIM), jnp.float32),
            jax.ShapeDtypeStruct((NPAIR, WREP), jnp.float32),
        ),
        scratch_types=[
            pltpu.VMEM((rows_pw,), jnp.int32),
            pltpu.VMEM((rows_pw,), jnp.int32),
            pltpu.VMEM((rows_pw, DIM), jnp.float32),
            pltpu.VMEM((rows_pw, WREP), jnp.float32),
            pltpu.VMEM((rows_pw, WREP), jnp.float32),
            pltpu.SemaphoreType.DMA,
            pltpu.SemaphoreType.DMA,
            pltpu.SemaphoreType.DMA,
            pltpu.SemaphoreType.DMA,
        ],
    )
    def k(se_hbm, so_hbm, x_hbm, w0_hbm, w1_hbm, out_hbm, wso_hbm,
          ie_v, io_v, rows_v, w0_v, w1_v, s0, s1, s2, s3):
        wid = lax.axis_index("s") * 2 + lax.axis_index("c")
        base = wid * rows_pw
        pltpu.sync_copy(se_hbm.at[pl.ds(base, rows_pw)], ie_v)
        pltpu.sync_copy(so_hbm.at[pl.ds(base, rows_pw)], io_v)
        pltpu.sync_copy(x_hbm.at[pl.ds(base, rows_pw)], rows_v)
        pltpu.sync_copy(w0_hbm.at[pl.ds(base, rows_pw)], w0_v)
        pltpu.sync_copy(w1_hbm.at[pl.ds(base, rows_pw)], w1_v)
        cp0 = pltpu.async_copy(rows_v, out_hbm.at[ie_v], s0)
        cp1 = pltpu.async_copy(rows_v, out_hbm.at[io_v], s1)
        cp2 = pltpu.async_copy(w0_v, wso_hbm.at[ie_v], s2)
        cp3 = pltpu.async_copy(w1_v, wso_hbm.at[io_v], s3)
        cp0.wait()
        cp1.wait()
        cp2.wait()
        cp3.wait()

    return k(slot_e, slot_o, xsrc, tw0r, tw1r)


def _ffn_block(x, W1, b1, W2, b2, W3, b3):
    h1 = jax.lax.dot_general(x, W1, NT, preferred_element_type=jnp.float32) + b1
    h3 = jax.lax.dot_general(x, W3, NT, preferred_element_type=jnp.float32) + b3
    p = h1 * h3
    h = p * jax.nn.sigmoid(p)
    return jax.lax.dot_general(h, W2, NT, preferred_element_type=jnp.float32) + b2


def _grouped_body(meta_ref, xs_ref, w1_ref, b1_ref, w2_ref, b2_ref,
                  w3_ref, b3_ref, wso_ref, ys_ref):
    i = pl.program_id(0)
    fv = meta_ref[i, 2]
    valid = meta_ref[i, 3]

    @pl.when(valid == 1)
    def _():
        o = _ffn_block(xs_ref[...], w1_ref[0], b1_ref[0], w2_ref[0],
                       b2_ref[0], w3_ref[0], b3_ref[0])
        row0 = meta_ref[i, 1] * BG
        rows = row0 + jax.lax.broadcasted_iota(jnp.int32, (BG, 1), 0)
        inseg = jnp.logical_and(rows >= meta_ref[i, 4], rows < meta_ref[i, 5])
        w = wso_ref[:, 0:1]
        contrib = jnp.where(inseg, o * w, 0.0)

        @pl.when(fv == 1)
        def _():
            ys_ref[...] = contrib

        @pl.when(fv == 0)
        def _():
            ys_ref[...] += contrib


def _grouped_moe(meta, xs, W1, b1, W2, b2, W3, b3, wso):
    grid_spec = pltpu.PrefetchScalarGridSpec(
        num_scalar_prefetch=1,
        grid=(G,),
        in_specs=[
            pl.BlockSpec((BG, DIM), lambda i, m: (m[i, 1], 0)),
            pl.BlockSpec((1, DFF, DIM), lambda i, m: (m[i, 0], 0, 0)),
            pl.BlockSpec((1, 1, DFF), lambda i, m: (m[i, 0], 0, 0)),
            pl.BlockSpec((1, DIM, DFF), lambda i, m: (m[i, 0], 0, 0)),
            pl.BlockSpec((1, 1, DIM), lambda i, m: (m[i, 0], 0, 0)),
            pl.BlockSpec((1, DFF, DIM), lambda i, m: (m[i, 0], 0, 0)),
            pl.BlockSpec((1, 1, DFF), lambda i, m: (m[i, 0], 0, 0)),
            pl.BlockSpec((BG, WREP), lambda i, m: (m[i, 1], 0)),
        ],
        out_specs=pl.BlockSpec((BG, DIM), lambda i, m: (m[i, 1], 0)),
    )
    return pl.pallas_call(
        _grouped_body,
        grid_spec=grid_spec,
        out_shape=jax.ShapeDtypeStruct((NPAIR, DIM), jnp.float32),
    )(meta, xs, W1, b1.reshape(E, 1, DFF), W2, b2.reshape(E, 1, DIM),
      W3, b3.reshape(E, 1, DFF), wso)


def _sc_combine(pos0, pos1, ys):
    mesh = plsc.VectorSubcoreMesh(core_axis_name="c", subcore_axis_name="s")
    rows_pw = CCH * CNCH

    @functools.partial(
        pl.kernel, mesh=mesh,
        out_type=jax.ShapeDtypeStruct((T, DIM), jnp.float32),
        scratch_types=[
            pltpu.VMEM((rows_pw,), jnp.int32),
            pltpu.VMEM((rows_pw,), jnp.int32),
            pltpu.VMEM((CCH, DIM), jnp.float32),
            pltpu.VMEM((CCH, DIM), jnp.float32),
            pltpu.SemaphoreType.DMA,
            pltpu.SemaphoreType.DMA,
        ],
    )
    def k(p0_hbm, p1_hbm, ys_hbm, out_hbm, i0_v, i1_v, r0_v, r1_v, s0, s1):
        wid = lax.axis_index("s") * 2 + lax.axis_index("c")
        base = wid * rows_pw
        pltpu.sync_copy(p0_hbm.at[pl.ds(base, rows_pw)], i0_v)
        pltpu.sync_copy(p1_hbm.at[pl.ds(base, rows_pw)], i1_v)
        for c in range(CNCH):
            cp0 = pltpu.async_copy(
                ys_hbm.at[i0_v.at[pl.ds(c * CCH, CCH)]], r0_v, s0)
            cp1 = pltpu.async_copy(
                ys_hbm.at[i1_v.at[pl.ds(c * CCH, CCH)]], r1_v, s1)
            cp0.wait()
            cp1.wait()

            def row(r, carry):
                for cc in range(DIM // 16):
                    sl = pl.ds(cc * 16, 16)
                    r0_v[r, sl] += r1_v[r, sl]
                return carry

            lax.fori_loop(0, CCH, row, 0)
            pltpu.sync_copy(r0_v, out_hbm.at[pl.ds(base + c * CCH, CCH)])

    return k(pos0, pos1, ys)


def _shared_body(x_ref, ws1_ref, bs1_ref, ws2_ref, bs2_ref, ws3_ref, bs3_ref,
                 ymoe_ref, y_ref):
    z = _ffn_block(x_ref[...], ws1_ref[...], bs1_ref[...], ws2_ref[...],
                   bs2_ref[...], ws3_ref[...], bs3_ref[...])
    y_ref[...] = z + ymoe_ref[...]


def _shared_ffn(x, Ws1, bs1, Ws2, bs2, Ws3, bs3, ymoe):
    zmap = lambda b: (0, 0)
    return pl.pallas_call(
        _shared_body,
        grid=(T // BM,),
        in_specs=[
            pl.BlockSpec((BM, DIM), lambda b: (b, 0)),
            pl.BlockSpec((DFF, DIM), zmap),
            pl.BlockSpec((1, DFF), zmap),
            pl.BlockSpec((DIM, DFF), zmap),
            pl.BlockSpec((1, DIM), zmap),
            pl.BlockSpec((DFF, DIM), zmap),
            pl.BlockSpec((1, DFF), zmap),
            pl.BlockSpec((BM, DIM), lambda b: (b, 0)),
        ],
        out_specs=pl.BlockSpec((BM, DIM), lambda b: (b, 0)),
        out_shape=jax.ShapeDtypeStruct((T, DIM), jnp.float32),
    )(x, Ws1, bs1.reshape(1, DFF), Ws2, bs2.reshape(1, DIM),
      Ws3, bs3.reshape(1, DFF), ymoe)


def kernel(x, Wg, expert_bias, W1, b1, W2, b2, W3, b3, Ws1, bs1, Ws2, bs2, Ws3, bs3):
    tw, idx, meta, se2, so2, tw0r, tw1r = _routing(x, Wg, expert_bias)
    slot_e = se2.reshape(T)
    slot_o = so2.reshape(T)
    xs, wso = _sc_dispatch(slot_e, slot_o, x, tw0r, tw1r)
    ys = _grouped_moe(meta, xs, W1, b1, W2, b2, W3, b3, wso)
    ymoe = _sc_combine(slot_e, slot_o, ys)
    y = _shared_ffn(x, Ws1, bs1, Ws2, bs2, Ws3, bs3, ymoe)
    return (y, tw, idx)
